# baseline (device time: 42654 ns/iter reference)
import jax
import jax.numpy as jnp
from jax import lax
from jax.experimental import pallas as pl
from jax.experimental.pallas import tpu as pltpu

N_DEV = 8
N_TOK = 256
D_IN = 128
D_OUT = 256
EXPERTS_PER_DEV = 2
CAPACITY = 12


def kernel(x, router_W, route_idx, expert_W):
    del router_W

    def body(x_ref, idx_ref, w_ref, out_ref, comm_ref, send_sems, recv_sems):
        my_i = lax.axis_index("i")
        right = lax.rem(my_i + 1, N_DEV)

        idx = idx_ref[:, :]
        row = lax.broadcasted_iota(jnp.int32, (N_TOK, N_TOK), 0)
        col = lax.broadcasted_iota(jnp.int32, (N_TOK, N_TOK), 1)
        tril = jnp.where(col <= row, 1.0, 0.0).astype(jnp.float32)

        acc = jnp.zeros((N_TOK, D_OUT), jnp.float32)
        for j in range(EXPERTS_PER_DEV):
            e = my_i * EXPERTS_PER_DEV + j
            match = idx == e
            match_f = jnp.where(match, 1.0, 0.0).astype(jnp.float32)
            rank = jnp.dot(tril, match_f, preferred_element_type=jnp.float32)
            keep = jnp.logical_and(match, rank <= float(CAPACITY))
            xm = jnp.where(keep, x_ref[:, :], 0.0)
            acc = acc + jnp.dot(
                xm, w_ref[j, :, :], preferred_element_type=jnp.float32
            )

        out_ref[:, :] = acc
        comm_ref[0, :, :] = acc

        for h in range(N_DEV - 1):
            rdma = pltpu.make_async_remote_copy(
                src_ref=comm_ref.at[h],
                dst_ref=comm_ref.at[h + 1],
                send_sem=send_sems.at[h],
                recv_sem=recv_sems.at[h],
                device_id=(right,),
                device_id_type=pl.DeviceIdType.MESH,
            )
            rdma.start()
            rdma.wait()
            out_ref[:, :] += comm_ref[h + 1, :, :]

    return pl.pallas_call(
        body,
        out_shape=jax.ShapeDtypeStruct((N_TOK, D_OUT), jnp.float32),
        in_specs=[
            pl.BlockSpec(memory_space=pltpu.VMEM),
            pl.BlockSpec(memory_space=pltpu.VMEM),
            pl.BlockSpec(memory_space=pltpu.VMEM),
        ],
        out_specs=pl.BlockSpec(memory_space=pltpu.VMEM),
        scratch_shapes=[
            pltpu.VMEM((N_DEV, N_TOK, D_OUT), jnp.float32),
            pltpu.SemaphoreType.DMA((N_DEV - 1,)),
            pltpu.SemaphoreType.DMA((N_DEV - 1,)),
        ],
    )(x, route_idx, expert_W)


# device time: 21111 ns/iter; 2.0205x vs baseline; 2.0205x over previous
import jax
import jax.numpy as jnp
from jax import lax
from jax.experimental import pallas as pl
from jax.experimental.pallas import tpu as pltpu

N_DEV = 8
N_TOK = 256
D_IN = 128
D_OUT = 256
EXPERTS_PER_DEV = 2
CAPACITY = 12


def kernel(x, router_W, route_idx, expert_W):
    del router_W

    def body(x_ref, idx_ref, w_ref, out_ref, send_ref, recv_ref,
             send_sems, recv_sems):
        my_i = lax.axis_index("i")

        idx = idx_ref[:, :]
        row = lax.broadcasted_iota(jnp.int32, (N_TOK, N_TOK), 0)
        col = lax.broadcasted_iota(jnp.int32, (N_TOK, N_TOK), 1)
        tril = jnp.where(col <= row, 1.0, 0.0).astype(jnp.float32)

        acc = jnp.zeros((N_TOK, D_OUT), jnp.float32)
        for j in range(EXPERTS_PER_DEV):
            e = my_i * EXPERTS_PER_DEV + j
            match = idx == e
            match_f = jnp.where(match, 1.0, 0.0).astype(jnp.float32)
            rank = jnp.dot(tril, match_f, preferred_element_type=jnp.float32)
            keep = jnp.logical_and(match, rank <= float(CAPACITY))
            xm = jnp.where(keep, x_ref[:, :], 0.0)
            acc = acc + jnp.dot(
                xm, w_ref[j, :, :], preferred_element_type=jnp.float32
            )

        partners = [my_i ^ (1 << r) for r in range(3)]

        barrier_sem = pltpu.get_barrier_semaphore()
        for p in partners:
            pl.semaphore_signal(
                barrier_sem, inc=1,
                device_id=(p,), device_id_type=pl.DeviceIdType.MESH,
            )
        pl.semaphore_wait(barrier_sem, len(partners))

        for r, p in enumerate(partners):
            send_ref[:, :] = acc
            rdma = pltpu.make_async_remote_copy(
                src_ref=send_ref,
                dst_ref=recv_ref.at[r],
                send_sem=send_sems.at[r],
                recv_sem=recv_sems.at[r],
                device_id=(p,),
                device_id_type=pl.DeviceIdType.MESH,
            )
            rdma.start()
            rdma.wait()
            acc = acc + recv_ref[r, :, :]

        out_ref[:, :] = acc

    return pl.pallas_call(
        body,
        out_shape=jax.ShapeDtypeStruct((N_TOK, D_OUT), jnp.float32),
        in_specs=[
            pl.BlockSpec(memory_space=pltpu.VMEM),
            pl.BlockSpec(memory_space=pltpu.VMEM),
            pl.BlockSpec(memory_space=pltpu.VMEM),
        ],
        out_specs=pl.BlockSpec(memory_space=pltpu.VMEM),
        scratch_shapes=[
            pltpu.VMEM((N_TOK, D_OUT), jnp.float32),
            pltpu.VMEM((3, N_TOK, D_OUT), jnp.float32),
            pltpu.SemaphoreType.DMA((3,)),
            pltpu.SemaphoreType.DMA((3,)),
        ],
        compiler_params=pltpu.CompilerParams(collective_id=0),
    )(x, route_idx, expert_W)


# device time: 15983 ns/iter; 2.6687x vs baseline; 1.3208x over previous
import jax
import jax.numpy as jnp
from jax import lax
from jax.experimental import pallas as pl
from jax.experimental.pallas import tpu as pltpu

N_DEV = 8
N_TOK = 256
D_IN = 128
D_OUT = 256
EXPERTS_PER_DEV = 2
CAPACITY = 12


def kernel(x, router_W, route_idx, expert_W):
    del router_W

    def body(x_ref, idx_ref, w_ref, out_ref, send_ref, recv_ref,
             send_sems, recv_sems):
        my_i = lax.axis_index("i")

        idx = idx_ref[:, :]
        row = lax.broadcasted_iota(jnp.int32, (N_TOK, N_TOK), 0)
        col = lax.broadcasted_iota(jnp.int32, (N_TOK, N_TOK), 1)
        tril = jnp.where(col <= row, 1.0, 0.0).astype(jnp.float32)

        ecol = lax.broadcasted_iota(jnp.int32, (N_TOK, EXPERTS_PER_DEV), 1)
        match = idx == my_i * EXPERTS_PER_DEV + ecol
        match_f = jnp.where(match, 1.0, 0.0).astype(jnp.float32)
        rank = jnp.dot(tril, match_f, preferred_element_type=jnp.float32)
        keep = jnp.logical_and(match, rank <= float(CAPACITY))
        xm = jnp.concatenate(
            [
                jnp.where(keep[:, j : j + 1], x_ref[:, :], 0.0)
                for j in range(EXPERTS_PER_DEV)
            ],
            axis=1,
        )
        w_cat = w_ref[:, :, :].reshape(EXPERTS_PER_DEV * D_IN, D_OUT)
        acc = jnp.dot(xm, w_cat, preferred_element_type=jnp.float32)

        partners = [my_i ^ m for m in (1, 3, 4)]

        barrier_sem = pltpu.get_barrier_semaphore()
        for p in partners:
            pl.semaphore_signal(
                barrier_sem, inc=1,
                device_id=(p,), device_id_type=pl.DeviceIdType.MESH,
            )
        pl.semaphore_wait(barrier_sem, len(partners))

        for r, p in enumerate(partners):
            send_ref[r, :, :] = acc.astype(jnp.bfloat16)
            rdma = pltpu.make_async_remote_copy(
                src_ref=send_ref.at[r],
                dst_ref=recv_ref.at[r],
                send_sem=send_sems.at[r],
                recv_sem=recv_sems.at[r],
                device_id=(p,),
                device_id_type=pl.DeviceIdType.MESH,
            )
            rdma.start()
            rdma.wait()
            acc = acc + recv_ref[r, :, :].astype(jnp.float32)

        out_ref[:, :] = acc

    return pl.pallas_call(
        body,
        out_shape=jax.ShapeDtypeStruct((N_TOK, D_OUT), jnp.float32),
        in_specs=[
            pl.BlockSpec(memory_space=pltpu.VMEM),
            pl.BlockSpec(memory_space=pltpu.VMEM),
            pl.BlockSpec(memory_space=pltpu.VMEM),
        ],
        out_specs=pl.BlockSpec(memory_space=pltpu.VMEM),
        scratch_shapes=[
            pltpu.VMEM((3, N_TOK, D_OUT), jnp.bfloat16),
            pltpu.VMEM((3, N_TOK, D_OUT), jnp.bfloat16),
            pltpu.SemaphoreType.DMA((3,)),
            pltpu.SemaphoreType.DMA((3,)),
        ],
        compiler_params=pltpu.CompilerParams(collective_id=0),
    )(x, route_idx, expert_W)


# device time: 10812 ns/iter; 3.9451x vs baseline; 1.4783x over previous
import jax
import jax.numpy as jnp
from jax import lax
from jax.experimental import pallas as pl
from jax.experimental.pallas import tpu as pltpu

N_DEV = 8
N_TOK = 256
D_IN = 128
D_OUT = 256
N_EXP = 16
EXPERTS_PER_DEV = 2
CAP = 12
SLOTS = EXPERTS_PER_DEV * CAP
GSLOTS = N_DEV * SLOTS


def kernel(x, router_W, route_idx, expert_W):
    del router_W

    idx_row = route_idx.reshape(1, N_TOK)
    cmod = (jnp.arange(GSLOTS, dtype=jnp.int32) % CAP).astype(
        jnp.float32
    ).reshape(1, GSLOTS)

    def body(x_ref, idxc_ref, idxr_ref, cmod_ref, w_ref, out_ref,
             allcomp, send_sems, recv_sems):
        my_i = lax.axis_index("i")

        barrier_sem = pltpu.get_barrier_semaphore()
        for k in range(1, N_DEV):
            pl.semaphore_signal(
                barrier_sem, inc=1,
                device_id=(my_i ^ k,), device_id_type=pl.DeviceIdType.MESH,
            )
        pl.semaphore_wait(barrier_sem, N_DEV - 1)

        idx_r = idxr_ref[:, :]
        jr = lax.broadcasted_iota(jnp.int32, (EXPERTS_PER_DEV, N_TOK), 0)
        matchT = idx_r == my_i * EXPERTS_PER_DEV + jr
        matchT_f = jnp.where(matchT, 1.0, 0.0).astype(jnp.float32)
        r_i = lax.broadcasted_iota(jnp.int32, (N_TOK, N_TOK), 0)
        c_i = lax.broadcasted_iota(jnp.int32, (N_TOK, N_TOK), 1)
        U = jnp.where(r_i <= c_i, 1.0, 0.0).astype(jnp.float32)
        rankT = jnp.dot(matchT_f, U, preferred_element_type=jnp.float32)
        s24 = lax.broadcasted_iota(jnp.int32, (SLOTS, EXPERTS_PER_DEV), 0)
        j24 = lax.broadcasted_iota(jnp.int32, (SLOTS, EXPERTS_PER_DEV), 1)
        E24 = jnp.where(
            (s24 >= CAP * j24) & (s24 < CAP * j24 + CAP), 1.0, 0.0
        ).astype(jnp.float32)
        rankT24 = jnp.dot(E24, rankT, preferred_element_type=jnp.float32)
        matchT24 = jnp.dot(E24, matchT_f, preferred_element_type=jnp.float32)
        srow = lax.broadcasted_iota(jnp.int32, (SLOTS, N_TOK), 0)
        rmod = jnp.where(srow < CAP, srow, srow - CAP).astype(jnp.float32)
        C = jnp.where(
            (matchT24 > 0.5) & (rankT24 == rmod + 1.0), 1.0, 0.0
        ).astype(jnp.float32)

        xc = jnp.dot(C, x_ref[:, :], preferred_element_type=jnp.float32)
        comp = jnp.concatenate(
            [
                jnp.dot(xc[:CAP, :], w_ref[0, :, :],
                        preferred_element_type=jnp.float32),
                jnp.dot(xc[CAP:, :], w_ref[1, :, :],
                        preferred_element_type=jnp.float32),
            ],
            axis=0,
        )
        allcomp[my_i, :, :] = comp

        sends = []
        for k in range(1, N_DEV):
            s = pltpu.make_async_remote_copy(
                src_ref=allcomp.at[my_i],
                dst_ref=allcomp.at[my_i],
                send_sem=send_sems.at[k],
                recv_sem=recv_sems.at[k],
                device_id=(my_i ^ k,),
                device_id_type=pl.DeviceIdType.MESH,
            )
            s.start()
            sends.append(s)

        idx_c = idxc_ref[:, :]
        e16 = lax.broadcasted_iota(jnp.int32, (N_TOK, N_EXP), 1)
        match_all = idx_c == e16
        match_all_f = jnp.where(match_all, 1.0, 0.0).astype(jnp.float32)
        tril = jnp.where(c_i <= r_i, 1.0, 0.0).astype(jnp.float32)
        rank_all = jnp.dot(tril, match_all_f,
                           preferred_element_type=jnp.float32)
        e_r = lax.broadcasted_iota(jnp.int32, (N_EXP, GSLOTS), 0)
        c_c = lax.broadcasted_iota(jnp.int32, (N_EXP, GSLOTS), 1)
        ET = jnp.where(
            (c_c >= CAP * e_r) & (c_c < CAP * e_r + CAP), 1.0, 0.0
        ).astype(jnp.float32)
        rank_exp = jnp.dot(rank_all, ET,
                           preferred_element_type=jnp.float32)
        match_exp = jnp.dot(match_all_f, ET,
                            preferred_element_type=jnp.float32)
        P = jnp.where(
            (match_exp > 0.5) & (rank_exp == cmod_ref[:, :] + 1.0), 1.0, 0.0
        ).astype(jnp.float32)

        for k in range(1, N_DEV):
            recv = pltpu.make_async_remote_copy(
                src_ref=allcomp.at[my_i],
                dst_ref=allcomp.at[my_i ^ k],
                send_sem=send_sems.at[k],
                recv_sem=recv_sems.at[k],
                device_id=(my_i ^ k,),
                device_id_type=pl.DeviceIdType.MESH,
            )
            recv.wait_recv()
        for s in sends:
            s.wait_send()

        allflat = allcomp[:, :, :].reshape(GSLOTS, D_OUT)
        out_ref[:, :] = jnp.dot(P, allflat,
                                preferred_element_type=jnp.float32)

    return pl.pallas_call(
        body,
        out_shape=jax.ShapeDtypeStruct((N_TOK, D_OUT), jnp.float32),
        in_specs=[
            pl.BlockSpec(memory_space=pltpu.VMEM),
            pl.BlockSpec(memory_space=pltpu.VMEM),
            pl.BlockSpec(memory_space=pltpu.VMEM),
            pl.BlockSpec(memory_space=pltpu.VMEM),
            pl.BlockSpec(memory_space=pltpu.VMEM),
        ],
        out_specs=pl.BlockSpec(memory_space=pltpu.VMEM),
        scratch_shapes=[
            pltpu.VMEM((N_DEV, SLOTS, D_OUT), jnp.float32),
            pltpu.SemaphoreType.DMA((N_DEV,)),
            pltpu.SemaphoreType.DMA((N_DEV,)),
        ],
        compiler_params=pltpu.CompilerParams(collective_id=0),
    )(x, route_idx, idx_row, cmod, expert_W)


# device time: 10451 ns/iter; 4.0813x vs baseline; 1.0345x over previous
import jax
import jax.numpy as jnp
from jax import lax
from jax.experimental import pallas as pl
from jax.experimental.pallas import tpu as pltpu

N_DEV = 8
N_TOK = 256
D_IN = 128
D_OUT = 256
N_EXP = 16
EXPERTS_PER_DEV = 2
CAP = 12
SLOTS = EXPERTS_PER_DEV * CAP
SLOTS_PAD = 32
GSLOTS = N_DEV * SLOTS_PAD


def kernel(x, router_W, route_idx, expert_W):
    del router_W

    idx_row = route_idx.reshape(1, N_TOK)
    c = jnp.arange(GSLOTS, dtype=jnp.int32)
    s = c % SLOTS_PAD
    e_of = (c // SLOTS_PAD) * EXPERTS_PER_DEV + s // CAP
    valid = s < SLOTS
    ET = (
        jnp.arange(N_EXP, dtype=jnp.int32)[:, None]
        == jnp.where(valid, e_of, -1)[None, :]
    ).astype(jnp.float32)
    cmod = jnp.where(valid, s % CAP, -2).astype(jnp.float32).reshape(1, GSLOTS)

    def body(x_ref, idxc_ref, idxr_ref, cmod_ref, et_ref, w_ref, out_ref,
             allcomp, send_sems, recv_sems):
        my_i = lax.axis_index("i")

        barrier_sem = pltpu.get_barrier_semaphore()
        for k in range(1, N_DEV):
            pl.semaphore_signal(
                barrier_sem, inc=1,
                device_id=(my_i ^ k,), device_id_type=pl.DeviceIdType.MESH,
            )
        pl.semaphore_wait(barrier_sem, N_DEV - 1)

        idx_r = idxr_ref[:, :]
        jr = lax.broadcasted_iota(jnp.int32, (EXPERTS_PER_DEV, N_TOK), 0)
        matchT = idx_r == my_i * EXPERTS_PER_DEV + jr
        matchT_f = jnp.where(matchT, 1.0, 0.0).astype(jnp.float32)
        r_i = lax.broadcasted_iota(jnp.int32, (N_TOK, N_TOK), 0)
        c_i = lax.broadcasted_iota(jnp.int32, (N_TOK, N_TOK), 1)
        U = jnp.where(r_i <= c_i, 1.0, 0.0).astype(jnp.float32)
        rankT = jnp.dot(matchT_f, U, preferred_element_type=jnp.float32)
        s32 = lax.broadcasted_iota(jnp.int32, (SLOTS_PAD, EXPERTS_PER_DEV), 0)
        j32 = lax.broadcasted_iota(jnp.int32, (SLOTS_PAD, EXPERTS_PER_DEV), 1)
        E32 = jnp.where(
            (s32 >= CAP * j32) & (s32 < CAP * j32 + CAP), 1.0, 0.0
        ).astype(jnp.float32)
        rankT32 = jnp.dot(E32, rankT, preferred_element_type=jnp.float32)
        matchT32 = jnp.dot(E32, matchT_f, preferred_element_type=jnp.float32)
        srow = lax.broadcasted_iota(jnp.int32, (SLOTS_PAD, N_TOK), 0)
        rmod = jnp.where(srow < CAP, srow, srow - CAP).astype(jnp.float32)
        C = jnp.where(
            (matchT32 > 0.5) & (rankT32 == rmod + 1.0), 1.0, 0.0
        ).astype(jnp.float32)

        xc = jnp.dot(C, x_ref[:, :], preferred_element_type=jnp.float32)
        comp = jnp.concatenate(
            [
                jnp.dot(xc[:CAP, :], w_ref[0, :, :],
                        preferred_element_type=jnp.float32),
                jnp.dot(xc[CAP:, :], w_ref[1, :, :],
                        preferred_element_type=jnp.float32),
            ],
            axis=0,
        )
        allcomp[my_i, :, :] = comp.astype(jnp.bfloat16)

        sends = []
        for k in range(1, N_DEV):
            s = pltpu.make_async_remote_copy(
                src_ref=allcomp.at[my_i],
                dst_ref=allcomp.at[my_i],
                send_sem=send_sems.at[k],
                recv_sem=recv_sems.at[k],
                device_id=(my_i ^ k,),
                device_id_type=pl.DeviceIdType.MESH,
            )
            s.start()
            sends.append(s)

        idx_c = idxc_ref[:, :]
        e16 = lax.broadcasted_iota(jnp.int32, (N_TOK, N_EXP), 1)
        match_all = idx_c == e16
        match_all_f = jnp.where(match_all, 1.0, 0.0).astype(jnp.float32)
        tril = jnp.where(c_i <= r_i, 1.0, 0.0).astype(jnp.float32)
        rank_all = jnp.dot(tril, match_all_f,
                           preferred_element_type=jnp.float32)
        rank_exp = jnp.dot(rank_all, et_ref[:, :],
                           preferred_element_type=jnp.float32)
        match_exp = jnp.dot(match_all_f, et_ref[:, :],
                            preferred_element_type=jnp.float32)
        P = jnp.where(
            (match_exp > 0.5) & (rank_exp == cmod_ref[:, :] + 1.0), 1.0, 0.0
        ).astype(jnp.bfloat16)

        for k in range(1, N_DEV):
            recv = pltpu.make_async_remote_copy(
                src_ref=allcomp.at[my_i],
                dst_ref=allcomp.at[my_i ^ k],
                send_sem=send_sems.at[k],
                recv_sem=recv_sems.at[k],
                device_id=(my_i ^ k,),
                device_id_type=pl.DeviceIdType.MESH,
            )
            recv.wait_recv()
        for s in sends:
            s.wait_send()

        allflat = allcomp[:, :, :].reshape(GSLOTS, D_OUT)
        out_ref[:, :] = jnp.dot(P, allflat,
                                preferred_element_type=jnp.float32)

    return pl.pallas_call(
        body,
        out_shape=jax.ShapeDtypeStruct((N_TOK, D_OUT), jnp.float32),
        in_specs=[pl.BlockSpec(memory_space=pltpu.VMEM)] * 6,
        out_specs=pl.BlockSpec(memory_space=pltpu.VMEM),
        scratch_shapes=[
            pltpu.VMEM((N_DEV, SLOTS_PAD, D_OUT), jnp.bfloat16),
            pltpu.SemaphoreType.DMA((N_DEV,)),
            pltpu.SemaphoreType.DMA((N_DEV,)),
        ],
        compiler_params=pltpu.CompilerParams(collective_id=0),
    )(x, route_idx, idx_row, cmod, ET, expert_W)


# device time: 9832 ns/iter; 4.3383x vs baseline; 1.0630x over previous
import jax
import jax.numpy as jnp
from jax import lax
from jax.experimental import pallas as pl
from jax.experimental.pallas import tpu as pltpu

N_DEV = 8
N_TOK = 256
D_IN = 128
D_OUT = 256
N_EXP = 16
EXPERTS_PER_DEV = 2
CAP = 12
SLOTS = EXPERTS_PER_DEV * CAP
SLOTS_PAD = 32
GSLOTS = N_DEV * SLOTS_PAD


def kernel(x, router_W, route_idx, expert_W):
    del router_W

    idx_row = route_idx.reshape(1, N_TOK)
    c = jnp.arange(GSLOTS, dtype=jnp.int32)
    s = c % SLOTS_PAD
    e_of = (c // SLOTS_PAD) * EXPERTS_PER_DEV + s // CAP
    valid = s < SLOTS
    ET = (
        jnp.arange(N_EXP, dtype=jnp.int32)[:, None]
        == jnp.where(valid, e_of, -1)[None, :]
    ).astype(jnp.float32)
    cmod = jnp.where(valid, s % CAP, -2).astype(jnp.float32).reshape(1, GSLOTS)

    def body(x_ref, idxc_ref, idxr_ref, cmod_ref, et_ref, w_ref, out_ref,
             allcomp, send_sems, recv_sems):
        my_i = lax.axis_index("i")

        barrier_sem = pltpu.get_barrier_semaphore()
        for k in range(1, N_DEV):
            pl.semaphore_signal(
                barrier_sem, inc=1,
                device_id=(my_i ^ k,), device_id_type=pl.DeviceIdType.MESH,
            )

        idx_r = idxr_ref[:, :]
        jr = lax.broadcasted_iota(jnp.int32, (EXPERTS_PER_DEV, N_TOK), 0)
        matchT = idx_r == my_i * EXPERTS_PER_DEV + jr
        matchT_f = jnp.where(matchT, 1.0, 0.0).astype(jnp.float32)
        r_i = lax.broadcasted_iota(jnp.int32, (N_TOK, N_TOK), 0)
        c_i = lax.broadcasted_iota(jnp.int32, (N_TOK, N_TOK), 1)
        U = jnp.where(r_i <= c_i, 1.0, 0.0).astype(jnp.float32)
        rankT = jnp.dot(matchT_f, U, preferred_element_type=jnp.float32)
        s32 = lax.broadcasted_iota(jnp.int32, (SLOTS_PAD, EXPERTS_PER_DEV), 0)
        j32 = lax.broadcasted_iota(jnp.int32, (SLOTS_PAD, EXPERTS_PER_DEV), 1)
        E32 = jnp.where(
            (s32 >= CAP * j32) & (s32 < CAP * j32 + CAP), 1.0, 0.0
        ).astype(jnp.float32)
        rankT32 = jnp.dot(E32, rankT, preferred_element_type=jnp.float32)
        matchT32 = jnp.dot(E32, matchT_f, preferred_element_type=jnp.float32)
        srow = lax.broadcasted_iota(jnp.int32, (SLOTS_PAD, N_TOK), 0)
        rmod = jnp.where(srow < CAP, srow, srow - CAP).astype(jnp.float32)
        C = jnp.where(
            (matchT32 > 0.5) & (rankT32 == rmod + 1.0), 1.0, 0.0
        ).astype(jnp.float32)

        xc = jnp.dot(C, x_ref[:, :], preferred_element_type=jnp.float32)
        comp = jnp.concatenate(
            [
                jnp.dot(xc[:CAP, :], w_ref[0, :, :],
                        preferred_element_type=jnp.float32),
                jnp.dot(xc[CAP:, :], w_ref[1, :, :],
                        preferred_element_type=jnp.float32),
            ],
            axis=0,
        )
        allcomp[my_i, :, :] = comp.astype(jnp.bfloat16)

        pl.semaphore_wait(barrier_sem, N_DEV - 1)
        sends = []
        for k in (6, 2, 5, 7, 1, 3, 4):
            s = pltpu.make_async_remote_copy(
                src_ref=allcomp.at[my_i],
                dst_ref=allcomp.at[my_i],
                send_sem=send_sems.at[k],
                recv_sem=recv_sems.at[k],
                device_id=(my_i ^ k,),
                device_id_type=pl.DeviceIdType.MESH,
            )
            s.start()
            sends.append(s)

        idx_c = idxc_ref[:, :]
        e16 = lax.broadcasted_iota(jnp.int32, (N_TOK, N_EXP), 1)
        match_all = idx_c == e16
        match_all_f = jnp.where(match_all, 1.0, 0.0).astype(jnp.float32)
        tril = jnp.where(c_i <= r_i, 1.0, 0.0).astype(jnp.float32)
        rank_all = jnp.dot(tril, match_all_f,
                           preferred_element_type=jnp.float32)
        rank_exp = jnp.dot(rank_all, et_ref[:, :],
                           preferred_element_type=jnp.float32)
        match_exp = jnp.dot(match_all_f, et_ref[:, :],
                            preferred_element_type=jnp.float32)
        P = jnp.where(
            (match_exp > 0.5) & (rank_exp == cmod_ref[:, :] + 1.0), 1.0, 0.0
        ).astype(jnp.bfloat16)

        for k in range(1, N_DEV):
            recv = pltpu.make_async_remote_copy(
                src_ref=allcomp.at[my_i],
                dst_ref=allcomp.at[my_i ^ k],
                send_sem=send_sems.at[k],
                recv_sem=recv_sems.at[k],
                device_id=(my_i ^ k,),
                device_id_type=pl.DeviceIdType.MESH,
            )
            recv.wait_recv()
        for s in sends:
            s.wait_send()

        allflat = allcomp[:, :, :].reshape(GSLOTS, D_OUT)
        out_ref[:, :] = jnp.dot(P, allflat,
                                preferred_element_type=jnp.float32)

    return pl.pallas_call(
        body,
        out_shape=jax.ShapeDtypeStruct((N_TOK, D_OUT), jnp.float32),
        in_specs=[pl.BlockSpec(memory_space=pltpu.VMEM)] * 6,
        out_specs=pl.BlockSpec(memory_space=pltpu.VMEM),
        scratch_shapes=[
            pltpu.VMEM((N_DEV, SLOTS_PAD, D_OUT), jnp.bfloat16),
            pltpu.SemaphoreType.DMA((N_DEV,)),
            pltpu.SemaphoreType.DMA((N_DEV,)),
        ],
        compiler_params=pltpu.CompilerParams(collective_id=0),
    )(x, route_idx, idx_row, cmod, ET, expert_W)
